# Initial kernel scaffold; baseline (speedup 1.0000x reference)
import jax
import jax.numpy as jnp
from jax.experimental import pallas as pl


def kernel(k, k_prime, exam_table, rel_table):
    raise NotImplementedError("write your pallas kernel here")


# speedup vs baseline: 288.1599x; 288.1599x over previous
import functools

import jax
import jax.numpy as jnp
from jax import lax
from jax.experimental import pallas as pl
from jax.experimental.pallas import tpu as pltpu
from jax.experimental.pallas import tpu_sc as plsc

BATCH = 16384
LIST = 200
TOTAL = BATCH * LIST
EXAM_N = 201
REL_N = 201 * 201

NC = 2
NS = 16
NW = NC * NS
L = 16

PER = TOTAL // NW
CHUNK = 4096
NCHUNK = PER // CHUNK


def _sigmoid_table(raw, sig, n):
    nv = (n + L - 1) // L
    last = n - L

    def body(j, carry):
        s = jnp.minimum(j * L, last)
        x = raw[pl.ds(s, L)]
        sig[pl.ds(s, L)] = 1.0 / (1.0 + jnp.exp(-x))
        return carry

    lax.fori_loop(0, nv, body, 0)


@functools.partial(
    pl.kernel,
    mesh=plsc.VectorSubcoreMesh(core_axis_name="c", subcore_axis_name="s"),
    out_type=jax.ShapeDtypeStruct((TOTAL,), jnp.float32),
    compiler_params=pltpu.CompilerParams(needs_layout_passes=False),
    scratch_types=[
        pltpu.VMEM((EXAM_N,), jnp.float32),
        pltpu.VMEM((EXAM_N,), jnp.float32),
        pltpu.VMEM((REL_N,), jnp.float32),
        pltpu.VMEM((REL_N,), jnp.float32),
        pltpu.VMEM((CHUNK,), jnp.int32),
        pltpu.VMEM((CHUNK,), jnp.int32),
        pltpu.VMEM((CHUNK,), jnp.float32),
    ],
)
def _all_pairs_pbm(k_hbm, kp_hbm, exam_hbm, rel_hbm, out_hbm,
                   exam_raw, exam_sig, rel_raw, rel_sig, k_v, kp_v, o_v):
    wid = lax.axis_index("s") * NC + lax.axis_index("c")
    base = wid * PER

    pltpu.sync_copy(exam_hbm, exam_raw)
    pltpu.sync_copy(rel_hbm, rel_raw)
    _sigmoid_table(exam_raw, exam_sig, EXAM_N)
    _sigmoid_table(rel_raw, rel_sig, REL_N)

    def vec_body(i, carry):
        kv = k_v[pl.ds(i * L, L)]
        kpv = kp_v[pl.ds(i * L, L)]
        e = plsc.load_gather(exam_sig, [kv])
        r = plsc.load_gather(rel_sig, [kv * kpv])
        o_v[pl.ds(i * L, L)] = e * r
        return carry

    for c in range(NCHUNK):
        cbase = base + c * CHUNK
        pltpu.sync_copy(k_hbm.at[pl.ds(cbase, CHUNK)], k_v)
        pltpu.sync_copy(kp_hbm.at[pl.ds(cbase, CHUNK)], kp_v)
        lax.fori_loop(0, CHUNK // L, vec_body, 0)
        pltpu.sync_copy(o_v, out_hbm.at[pl.ds(cbase, CHUNK)])


def kernel(k, k_prime, exam_table, rel_table):
    kf = k.reshape(TOTAL).astype(jnp.int32)
    kpf = k_prime.reshape(TOTAL).astype(jnp.int32)
    out = _all_pairs_pbm(kf, kpf,
                         exam_table.reshape(EXAM_N),
                         rel_table.reshape(REL_N))
    return out.reshape(BATCH, LIST)


# speedup vs baseline: 422.3421x; 1.4657x over previous
import functools

import jax
import jax.numpy as jnp
from jax import lax
from jax.experimental import pallas as pl
from jax.experimental.pallas import tpu as pltpu
from jax.experimental.pallas import tpu_sc as plsc

BATCH = 16384
LIST = 200
TOTAL = BATCH * LIST
EXAM_N = 201
REL_N = 201 * 201

NC = 2
NS = 16
NW = NC * NS
L = 16

PER = TOTAL // NW
CHUNK = 12800
NCHUNK = PER // CHUNK
VECS = CHUNK // L


def _sigmoid(x):
    return 1.0 / (1.0 + jnp.exp(-x))


def _sigmoid_table_inplace(buf, n, unroll):
    nfull = n // L
    tail_raw = buf[pl.ds(n - L, L)]

    @plsc.parallel_loop(0, nfull, 1, unroll=unroll)
    def _(j):
        x = buf[pl.ds(j * L, L)]
        buf[pl.ds(j * L, L)] = _sigmoid(x)

    buf[pl.ds(n - L, L)] = _sigmoid(tail_raw)


@functools.partial(
    pl.kernel,
    mesh=plsc.VectorSubcoreMesh(core_axis_name="c", subcore_axis_name="s"),
    out_type=jax.ShapeDtypeStruct((TOTAL,), jnp.float32),
    compiler_params=pltpu.CompilerParams(needs_layout_passes=False),
    scratch_types=[
        pltpu.VMEM((EXAM_N,), jnp.float32),
        pltpu.VMEM((REL_N,), jnp.float32),
        pltpu.VMEM((CHUNK,), jnp.int32),
        pltpu.VMEM((CHUNK,), jnp.int32),
        pltpu.VMEM((CHUNK,), jnp.int32),
        pltpu.VMEM((CHUNK,), jnp.int32),
        pltpu.VMEM((CHUNK,), jnp.float32),
        pltpu.VMEM((CHUNK,), jnp.float32),
        pltpu.SemaphoreType.DMA,
        pltpu.SemaphoreType.DMA,
        pltpu.SemaphoreType.DMA,
        pltpu.SemaphoreType.DMA,
    ],
)
def _all_pairs_pbm(k_hbm, kp_hbm, exam_hbm, rel_hbm, out_hbm,
                   exam_t, rel_t, k0, k1, kp0, kp1, o0, o1,
                   isem0, isem1, osem0, osem1):
    wid = lax.axis_index("s") * NC + lax.axis_index("c")
    base = wid * PER
    k_v = (k0, k1)
    kp_v = (kp0, kp1)
    o_v = (o0, o1)
    isem = (isem0, isem1)
    osem = (osem0, osem1)

    def start_in(c):
        s = c % 2
        cbase = base + c * CHUNK
        hk = pltpu.async_copy(k_hbm.at[pl.ds(cbase, CHUNK)], k_v[s], isem[s])
        hkp = pltpu.async_copy(kp_hbm.at[pl.ds(cbase, CHUNK)], kp_v[s], isem[s])
        return hk, hkp

    in_flight = {0: start_in(0)}
    pltpu.sync_copy(exam_hbm, exam_t)
    pltpu.sync_copy(rel_hbm, rel_t)
    _sigmoid_table_inplace(exam_t, EXAM_N, unroll=4)
    _sigmoid_table_inplace(rel_t, REL_N, unroll=8)

    out_flight = {}
    for c in range(NCHUNK):
        s = c % 2
        hk, hkp = in_flight.pop(c)
        hk.wait()
        hkp.wait()
        if c + 1 < NCHUNK:
            in_flight[c + 1] = start_in(c + 1)
        if c >= 2:
            out_flight.pop(c - 2).wait()

        kb, kpb, ob = k_v[s], kp_v[s], o_v[s]

        @plsc.parallel_loop(0, VECS, 1, unroll=8)
        def _(i):
            kv = kb[pl.ds(i * L, L)]
            kpv = kpb[pl.ds(i * L, L)]
            e = plsc.load_gather(exam_t, [kv])
            r = plsc.load_gather(rel_t, [kv * kpv])
            ob[pl.ds(i * L, L)] = e * r

        cbase = base + c * CHUNK
        out_flight[c] = pltpu.async_copy(ob, out_hbm.at[pl.ds(cbase, CHUNK)],
                                         osem[s])
    for h in out_flight.values():
        h.wait()


def kernel(k, k_prime, exam_table, rel_table):
    kf = k.reshape(TOTAL).astype(jnp.int32)
    kpf = k_prime.reshape(TOTAL).astype(jnp.int32)
    out = _all_pairs_pbm(kf, kpf,
                         exam_table.reshape(EXAM_N),
                         rel_table.reshape(REL_N))
    return out.reshape(BATCH, LIST)


# speedup vs baseline: 652.2235x; 1.5443x over previous
import jax
import jax.numpy as jnp
from jax import lax
from jax.experimental import pallas as pl
from jax.experimental.pallas import tpu as pltpu
from jax.experimental.pallas import tpu_sc as plsc

BATCH = 16384
LIST = 200
EXAM_N = 201
REL_N = 201 * 201

NC = 2
NS = 16
NW = NC * NS
L = 16

ROWS_PER_W = BATCH // NW
RB = 32
NCHUNK = ROWS_PER_W // RB

COL_OFFS = tuple(range(0, LIST - L + 1, L)) + (LIST - L,)


def _sigmoid(x):
    return 1.0 / (1.0 + jnp.exp(-x))


def _sigmoid_table_inplace(buf, n, unroll):
    nfull = n // L
    tail_raw = buf[pl.ds(n - L, L)]

    @plsc.parallel_loop(0, nfull, 1, unroll=unroll)
    def _(j):
        x = buf[pl.ds(j * L, L)]
        buf[pl.ds(j * L, L)] = _sigmoid(x)

    buf[pl.ds(n - L, L)] = _sigmoid(tail_raw)


@pl.kernel(
    mesh=plsc.VectorSubcoreMesh(core_axis_name="c", subcore_axis_name="s"),
    out_type=jax.ShapeDtypeStruct((BATCH, LIST), jnp.float32),
    compiler_params=pltpu.CompilerParams(needs_layout_passes=False),
    scratch_types=[
        pltpu.VMEM((EXAM_N,), jnp.float32),
        pltpu.VMEM((REL_N,), jnp.float32),
        pltpu.VMEM((RB, LIST), jnp.int32),
        pltpu.VMEM((RB, LIST), jnp.int32),
        pltpu.VMEM((RB, LIST), jnp.int32),
        pltpu.VMEM((RB, LIST), jnp.int32),
        pltpu.VMEM((RB, LIST), jnp.float32),
        pltpu.VMEM((RB, LIST), jnp.float32),
        pltpu.SemaphoreType.DMA,
        pltpu.SemaphoreType.DMA,
        pltpu.SemaphoreType.DMA,
        pltpu.SemaphoreType.DMA,
    ],
)
def _all_pairs_pbm(k_hbm, kp_hbm, exam_hbm, rel_hbm, out_hbm,
                   exam_t, rel_t, k0, k1, kp0, kp1, o0, o1,
                   isem0, isem1, osem0, osem1):
    wid = lax.axis_index("s") * NC + lax.axis_index("c")
    row_base = wid * ROWS_PER_W
    k_v = (k0, k1)
    kp_v = (kp0, kp1)
    o_v = (o0, o1)
    isem = (isem0, isem1)
    osem = (osem0, osem1)

    def start_in(c):
        s = c % 2
        r0 = row_base + c * RB
        hk = pltpu.async_copy(k_hbm.at[pl.ds(r0, RB)], k_v[s], isem[s])
        hkp = pltpu.async_copy(kp_hbm.at[pl.ds(r0, RB)], kp_v[s], isem[s])
        return hk, hkp

    in_flight = {0: start_in(0)}
    pltpu.sync_copy(exam_hbm, exam_t)
    pltpu.sync_copy(rel_hbm, rel_t)
    _sigmoid_table_inplace(exam_t, EXAM_N, unroll=4)
    _sigmoid_table_inplace(rel_t, REL_N, unroll=8)

    out_flight = {}
    for c in range(NCHUNK):
        s = c % 2
        hk, hkp = in_flight.pop(c)
        hk.wait()
        hkp.wait()
        if c + 1 < NCHUNK:
            in_flight[c + 1] = start_in(c + 1)
        if c >= 2:
            out_flight.pop(c - 2).wait()

        kb, kpb, ob = k_v[s], kp_v[s], o_v[s]

        @plsc.parallel_loop(0, RB, 1, unroll=1)
        def _(r):
            for col in COL_OFFS:
                kv = kb[r, pl.ds(col, L)]
                kpv = kpb[r, pl.ds(col, L)]
                e = plsc.load_gather(exam_t, [kv])
                g = plsc.load_gather(rel_t, [kv * kpv])
                ob[r, pl.ds(col, L)] = e * g

        r0 = row_base + c * RB
        out_flight[c] = pltpu.async_copy(ob, out_hbm.at[pl.ds(r0, RB)],
                                         osem[s])
    for h in out_flight.values():
        h.wait()


def kernel(k, k_prime, exam_table, rel_table):
    return _all_pairs_pbm(k.astype(jnp.int32), k_prime.astype(jnp.int32),
                          exam_table.reshape(EXAM_N),
                          rel_table.reshape(REL_N))


# speedup vs baseline: 1078.5632x; 1.6537x over previous
import jax
import jax.numpy as jnp
from jax import lax
from jax.experimental import pallas as pl
from jax.experimental.pallas import tpu as pltpu
from jax.experimental.pallas import tpu_sc as plsc

BATCH = 16384
LIST = 200
EXAM_N = 201
REL_N = 201 * 201

NC = 2
NS = 16
NW = NC * NS
L = 16

COLS_PER_W = BATCH // NW
RG = 8
NCHUNK = LIST // RG
SLICES = RG * COLS_PER_W // L


def _sigmoid(x):
    return 1.0 / (1.0 + jnp.exp(-x))


def _sigmoid_table_inplace(buf, n, unroll):
    nfull = n // L
    tail_raw = buf[pl.ds(n - L, L)]

    @plsc.parallel_loop(0, nfull, 1, unroll=unroll)
    def _(j):
        x = buf[pl.ds(j * L, L)]
        buf[pl.ds(j * L, L)] = _sigmoid(x)

    buf[pl.ds(n - L, L)] = _sigmoid(tail_raw)


@pl.kernel(
    mesh=plsc.VectorSubcoreMesh(core_axis_name="c", subcore_axis_name="s"),
    out_type=jax.ShapeDtypeStruct((LIST, BATCH), jnp.float32),
    compiler_params=pltpu.CompilerParams(needs_layout_passes=False),
    scratch_types=[
        pltpu.VMEM((EXAM_N,), jnp.float32),
        pltpu.VMEM((REL_N,), jnp.float32),
        pltpu.VMEM((RG, COLS_PER_W), jnp.int32),
        pltpu.VMEM((RG, COLS_PER_W), jnp.int32),
        pltpu.VMEM((RG, COLS_PER_W), jnp.int32),
        pltpu.VMEM((RG, COLS_PER_W), jnp.int32),
        pltpu.VMEM((RG, COLS_PER_W), jnp.float32),
        pltpu.VMEM((RG, COLS_PER_W), jnp.float32),
        pltpu.SemaphoreType.DMA,
        pltpu.SemaphoreType.DMA,
        pltpu.SemaphoreType.DMA,
        pltpu.SemaphoreType.DMA,
    ],
)
def _all_pairs_pbm(k_hbm, kp_hbm, exam_hbm, rel_hbm, out_hbm,
                   exam_t, rel_t, k0, k1, kp0, kp1, o0, o1,
                   isem0, isem1, osem0, osem1):
    wid = lax.axis_index("s") * NC + lax.axis_index("c")
    col_base = wid * COLS_PER_W
    k_v = (k0, k1)
    kp_v = (kp0, kp1)
    o_v = (o0, o1)
    isem = (isem0, isem1)
    osem = (osem0, osem1)

    def start_in(c):
        s = c % 2
        r0 = c * RG
        hk = pltpu.async_copy(
            k_hbm.at[pl.ds(r0, RG), pl.ds(col_base, COLS_PER_W)], k_v[s], isem[s])
        hkp = pltpu.async_copy(
            kp_hbm.at[pl.ds(r0, RG), pl.ds(col_base, COLS_PER_W)], kp_v[s], isem[s])
        return hk, hkp

    in_flight = {0: start_in(0)}
    pltpu.sync_copy(exam_hbm, exam_t)
    pltpu.sync_copy(rel_hbm, rel_t)
    _sigmoid_table_inplace(exam_t, EXAM_N, unroll=4)
    _sigmoid_table_inplace(rel_t, REL_N, unroll=8)

    out_flight = {}
    for c in range(NCHUNK):
        s = c % 2
        hk, hkp = in_flight.pop(c)
        hk.wait()
        hkp.wait()
        if c + 1 < NCHUNK:
            in_flight[c + 1] = start_in(c + 1)
        if c >= 2:
            out_flight.pop(c - 2).wait()

        kb, kpb, ob = k_v[s], kp_v[s], o_v[s]

        @plsc.parallel_loop(0, SLICES, 1, unroll=8)
        def _(i):
            r = lax.shift_right_logical(i, 5)
            col = (i & 31) * L
            kv = kb[r, pl.ds(col, L)]
            kpv = kpb[r, pl.ds(col, L)]
            e = plsc.load_gather(exam_t, [kv])
            g = plsc.load_gather(rel_t, [kv * kpv])
            ob[r, pl.ds(col, L)] = e * g

        r0 = c * RG
        out_flight[c] = pltpu.async_copy(
            ob, out_hbm.at[pl.ds(r0, RG), pl.ds(col_base, COLS_PER_W)], osem[s])
    for h in out_flight.values():
        h.wait()


def kernel(k, k_prime, exam_table, rel_table):
    out_t = _all_pairs_pbm(k.astype(jnp.int32).T, k_prime.astype(jnp.int32).T,
                           exam_table.reshape(EXAM_N),
                           rel_table.reshape(REL_N))
    return out_t.T
